# baseline (speedup 1.0000x reference)
import functools

import jax
import jax.numpy as jnp
from jax.experimental import pallas as pl
from jax.experimental.pallas import tpu as pltpu

AUX_COEF = 0.01
Z_COEF = 0.001
BLK = 1024
KC = 2


def _router_body(x_ref, w_ref, i0_ref, i1_ref, w0_ref, w1_ref,
                 cnt_ref, ps_ref, lse_ref, lg_acc, ps_acc, cnt_acc,
                 lse_acc, *, n_experts, nb):
    i = pl.program_id(0)
    k = pl.program_id(1)
    part = jnp.dot(x_ref[...], w_ref[...],
                   preferred_element_type=jnp.float32)

    @pl.when(k == 0)
    def _set():
        lg_acc[...] = part

    @pl.when(k > 0)
    def _add():
        lg_acc[...] += part

    @pl.when(k == KC - 1)
    def _post():
        lt = lg_acc[...].T
        iota = jax.lax.broadcasted_iota(jnp.int32, lt.shape, 0)

        m0 = jnp.max(lt, axis=0, keepdims=True)
        i0 = jnp.min(jnp.where(lt == m0, iota, n_experts), axis=0,
                     keepdims=True)
        masked = jnp.where(iota == i0, jnp.float32(-1e30), lt)
        m1 = jnp.max(masked, axis=0, keepdims=True)
        i1 = jnp.min(jnp.where(masked == m1, iota, n_experts), axis=0,
                     keepdims=True)

        e1 = jnp.exp(m1 - m0)
        denom = 1.0 + e1
        w0_ref[...] = (1.0 / denom)[None]
        w1_ref[...] = (e1 / denom)[None]
        i0_ref[...] = i0[None]
        i1_ref[...] = i1[None]

        ex = jnp.exp(lt - m0)
        ssum = jnp.sum(ex, axis=0, keepdims=True)
        probs = ex * (1.0 / ssum)
        one_hot = ((iota == i0).astype(jnp.float32)
                   + (iota == i1).astype(jnp.float32))
        lse_row = m0 + jnp.log(ssum)

        @pl.when(i == 0)
        def _init():
            ps_acc[...] = probs
            cnt_acc[...] = one_hot
            lse_acc[...] = lse_row

        @pl.when(i > 0)
        def _accum():
            ps_acc[...] += probs
            cnt_acc[...] += one_hot
            lse_acc[...] += lse_row

        @pl.when(i == nb - 1)
        def _finish():
            cnt_ref[...] = jnp.sum(cnt_acc[...], axis=1, keepdims=True)
            ps_ref[...] = jnp.sum(ps_acc[...], axis=1, keepdims=True)
            lse_ref[...] = jnp.sum(lse_acc[...], axis=1, keepdims=True)


def kernel(x, W):
    B, S, D = x.shape
    E = W.shape[1]
    N = B * S
    nb = N // BLK
    dc = D // KC
    x2 = x.reshape(N, D)

    body = functools.partial(_router_body, n_experts=E, nb=nb)
    i0, i1, w0, w1, cnt, ps, lse = pl.pallas_call(
        body,
        grid=(nb, KC),
        in_specs=[
            pl.BlockSpec((BLK, dc), lambda i, k: (i, k)),
            pl.BlockSpec((dc, E), lambda i, k: (k, 0)),
        ],
        out_specs=[
            pl.BlockSpec((1, 1, BLK), lambda i, k: (i, 0, 0)),
            pl.BlockSpec((1, 1, BLK), lambda i, k: (i, 0, 0)),
            pl.BlockSpec((1, 1, BLK), lambda i, k: (i, 0, 0)),
            pl.BlockSpec((1, 1, BLK), lambda i, k: (i, 0, 0)),
            pl.BlockSpec((E, 1), lambda i, k: (0, 0)),
            pl.BlockSpec((E, 1), lambda i, k: (0, 0)),
            pl.BlockSpec((1, 1), lambda i, k: (0, 0)),
        ],
        out_shape=[
            jax.ShapeDtypeStruct((nb, 1, BLK), jnp.int32),
            jax.ShapeDtypeStruct((nb, 1, BLK), jnp.int32),
            jax.ShapeDtypeStruct((nb, 1, BLK), jnp.float32),
            jax.ShapeDtypeStruct((nb, 1, BLK), jnp.float32),
            jax.ShapeDtypeStruct((E, 1), jnp.float32),
            jax.ShapeDtypeStruct((E, 1), jnp.float32),
            jax.ShapeDtypeStruct((1, 1), jnp.float32),
        ],
        scratch_shapes=[
            pltpu.VMEM((BLK, E), jnp.float32),
            pltpu.VMEM((E, BLK), jnp.float32),
            pltpu.VMEM((E, BLK), jnp.float32),
            pltpu.VMEM((1, BLK), jnp.float32),
        ],
    )(x2, W)

    idx = jnp.stack([i0.reshape(N), i1.reshape(N)], axis=-1).reshape(B, S, 2)
    wts = jnp.stack([w0.reshape(N), w1.reshape(N)], axis=-1).reshape(B, S, 2)
    tokens_per_expert = cnt[:, 0] / N
    router_prob_per_expert = ps[:, 0] / N
    balance_loss = jnp.sum(tokens_per_expert * router_prob_per_expert) * E
    z_loss = (lse[0, 0] / N) ** 2
    return (idx, wts, balance_loss * AUX_COEF, z_loss * Z_COEF,
            tokens_per_expert)


# speedup vs baseline: 1.3153x; 1.3153x over previous
import functools

import jax
import jax.numpy as jnp
from jax.experimental import pallas as pl
from jax.experimental.pallas import tpu as pltpu

AUX_COEF = 0.01
Z_COEF = 0.001
BLK = 1024


def _router_body(x_ref, w_ref, i0_ref, i1_ref, w0_ref, w1_ref,
                 cnt_ref, ps_ref, lse_ref, ps_acc, cnt_acc, lse_acc,
                 *, n_experts, nb):
    i = pl.program_id(0)
    logits = jnp.dot(x_ref[...], w_ref[...],
                     preferred_element_type=jnp.float32)
    lt = logits.T
    iota = jax.lax.broadcasted_iota(jnp.int32, lt.shape, 0)

    m0 = jnp.max(lt, axis=0, keepdims=True)
    i0 = jnp.min(jnp.where(lt == m0, iota, n_experts), axis=0,
                 keepdims=True)
    masked = jnp.where(iota == i0, jnp.float32(-1e30), lt)
    m1 = jnp.max(masked, axis=0, keepdims=True)
    i1 = jnp.min(jnp.where(masked == m1, iota, n_experts), axis=0,
                 keepdims=True)

    e1 = jnp.exp(m1 - m0)
    denom = 1.0 + e1
    w0_ref[...] = (1.0 / denom)[None]
    w1_ref[...] = (e1 / denom)[None]
    i0_ref[...] = i0[None]
    i1_ref[...] = i1[None]

    ex = jnp.exp(lt - m0)
    ssum = jnp.sum(ex, axis=0, keepdims=True)
    probs = ex * (1.0 / ssum)
    one_hot = ((iota == i0).astype(jnp.float32)
               + (iota == i1).astype(jnp.float32))
    lse_row = m0 + jnp.log(ssum)

    @pl.when(i == 0)
    def _init():
        ps_acc[...] = probs
        cnt_acc[...] = one_hot
        lse_acc[...] = lse_row

    @pl.when(i > 0)
    def _accum():
        ps_acc[...] += probs
        cnt_acc[...] += one_hot
        lse_acc[...] += lse_row

    @pl.when(i == nb - 1)
    def _finish():
        cnt_ref[...] = jnp.sum(cnt_acc[...], axis=1, keepdims=True)
        ps_ref[...] = jnp.sum(ps_acc[...], axis=1, keepdims=True)
        lse_ref[...] = jnp.sum(lse_acc[...], axis=1, keepdims=True)


def kernel(x, W):
    B, S, D = x.shape
    E = W.shape[1]
    N = B * S
    nb = N // BLK
    x2 = x.reshape(N, D)

    body = functools.partial(_router_body, n_experts=E, nb=nb)
    i0, i1, w0, w1, cnt, ps, lse = pl.pallas_call(
        body,
        grid=(nb,),
        in_specs=[
            pl.BlockSpec((BLK, D), lambda i: (i, 0)),
            pl.BlockSpec((D, E), lambda i: (0, 0)),
        ],
        out_specs=[
            pl.BlockSpec((1, 1, BLK), lambda i: (i, 0, 0)),
            pl.BlockSpec((1, 1, BLK), lambda i: (i, 0, 0)),
            pl.BlockSpec((1, 1, BLK), lambda i: (i, 0, 0)),
            pl.BlockSpec((1, 1, BLK), lambda i: (i, 0, 0)),
            pl.BlockSpec((E, 1), lambda i: (0, 0)),
            pl.BlockSpec((E, 1), lambda i: (0, 0)),
            pl.BlockSpec((1, 1), lambda i: (0, 0)),
        ],
        out_shape=[
            jax.ShapeDtypeStruct((nb, 1, BLK), jnp.int32),
            jax.ShapeDtypeStruct((nb, 1, BLK), jnp.int32),
            jax.ShapeDtypeStruct((nb, 1, BLK), jnp.float32),
            jax.ShapeDtypeStruct((nb, 1, BLK), jnp.float32),
            jax.ShapeDtypeStruct((E, 1), jnp.float32),
            jax.ShapeDtypeStruct((E, 1), jnp.float32),
            jax.ShapeDtypeStruct((1, 1), jnp.float32),
        ],
        scratch_shapes=[
            pltpu.VMEM((E, BLK), jnp.float32),
            pltpu.VMEM((E, BLK), jnp.float32),
            pltpu.VMEM((1, BLK), jnp.float32),
        ],
    )(x2, W)

    idx = jnp.stack([i0.reshape(N), i1.reshape(N)], axis=-1).reshape(B, S, 2)
    wts = jnp.stack([w0.reshape(N), w1.reshape(N)], axis=-1).reshape(B, S, 2)
    tokens_per_expert = cnt[:, 0] / N
    router_prob_per_expert = ps[:, 0] / N
    balance_loss = jnp.sum(tokens_per_expert * router_prob_per_expert) * E
    z_loss = (lse[0, 0] / N) ** 2
    return (idx, wts, balance_loss * AUX_COEF, z_loss * Z_COEF,
            tokens_per_expert)
